# baseline (speedup 1.0000x reference)
import functools

import jax
import jax.numpy as jnp
from jax import lax
from jax.experimental import pallas as pl
from jax.experimental.pallas import tpu as pltpu
from jax.experimental.pallas import tpu_sc as plsc

N = 10000
E = 320000
NP = 10112
NW = 32
BLK = 128
KCH = -(-E // (NW * BLK))
EP = NW * BLK * KCH
EW_ = KCH * BLK

NBLK = -(-E // BLK)
KA = 56
KB = -(-(NBLK - 16 * KA) // 16)
KMAX = max(KA, KB)
RP = NP // 16
D = 128
L = 16

_mesh = plsc.VectorSubcoreMesh(core_axis_name="c", subcore_axis_name="s")


@functools.partial(
    pl.kernel,
    mesh=_mesh,
    out_type=jax.ShapeDtypeStruct((2, NP, D), jnp.float32),
    scratch_types=[
        pltpu.VMEM((KMAX, BLK), jnp.int32),
        pltpu.VMEM((KMAX, BLK), jnp.int32),
        pltpu.VMEM((BLK, D), jnp.float32),
        pltpu.VMEM_SHARED((NP, D), jnp.float32),
        pltpu.SemaphoreType.DMA,
    ],
)
def _edge_scatter(table_hbm, src_hbm, dst_hbm, zeros_hbm, out_hbm,
                  src_v, dst_v, rows_v, acc_sh, sem):
    c = lax.axis_index("c")
    s = lax.axis_index("s")
    w = c * 16 + s
    pltpu.sync_copy(zeros_hbm, acc_sh.at[pl.ds(s * RP, RP)])
    pltpu.sync_copy(src_hbm.at[w], src_v)
    pltpu.sync_copy(dst_hbm.at[w], dst_v)
    plsc.subcore_barrier()

    def body(j, carry):
        pltpu.async_copy(table_hbm.at[src_v.at[j]], rows_v, sem).wait()
        pltpu.sync_copy(rows_v, acc_sh.at[dst_v.at[j]], add=True)
        return carry

    @pl.when(c == 0)
    def _():
        lax.fori_loop(0, KA, body, 0)

    @pl.when(c == 1)
    def _():
        lax.fori_loop(0, KB, body, 0)
    plsc.subcore_barrier()
    pltpu.sync_copy(acc_sh.at[pl.ds(s * RP, RP)],
                    out_hbm.at[c, pl.ds(s * RP, RP)])


@functools.partial(
    pl.kernel,
    mesh=_mesh,
    out_type=jax.ShapeDtypeStruct((NW, NP), jnp.float32),
    scratch_types=[
        pltpu.VMEM((NP,), jnp.float32),
        pltpu.VMEM((NP,), jnp.float32),
        pltpu.VMEM((EW_,), jnp.int32),
        pltpu.VMEM((EW_,), jnp.int32),
    ],
    compiler_params=pltpu.CompilerParams(needs_layout_passes=False),
)
def _deg_scatter(ew_hbm, srcf_hbm, dstf_hbm, zrow_hbm, out_hbm,
                 ew_v, acc_v, src_v, dst_v):
    c = lax.axis_index("c")
    s = lax.axis_index("s")
    w = c * 16 + s
    pltpu.sync_copy(ew_hbm, ew_v)
    pltpu.sync_copy(zrow_hbm, acc_v)
    pltpu.sync_copy(srcf_hbm.at[w], src_v)
    pltpu.sync_copy(dstf_hbm.at[w], dst_v)

    def body(g, carry):
        idx_s = src_v[pl.ds(g * L, L)]
        idx_d = dst_v[pl.ds(g * L, L)]
        vals = plsc.load_gather(ew_v, [idx_s])
        plsc.addupdate_scatter(acc_v, [idx_d], vals)
        return carry

    lax.fori_loop(0, EW_ // L, body, 0)
    pltpu.sync_copy(acc_v, out_hbm.at[w])


def _tc_first(deg_ref, ew_ref, x_ref, w_ref, hs_ref, h_ref, f_ref, g_ref, s_ref):
    deg = 1.0 + jnp.sum(deg_ref[...], axis=0)[:, None]
    dinv = jnp.where(deg > 0, lax.rsqrt(deg), 0.0)
    f = dinv * ew_ref[...]
    h = jnp.dot(x_ref[...], w_ref[...], preferred_element_type=jnp.float32)
    hs_ref[...] = h * f
    h_ref[...] = h
    f_ref[...] = f
    g_ref[...] = dinv
    s_ref[...] = dinv * dinv


def _tc_mid(acc_ref, h_ref, f_ref, g_ref, s_ref, b_ref, w_ref, hs_o, h_o):
    accp = acc_ref[...]
    z = g_ref[...] * (accp[0] + accp[1]) + s_ref[...] * h_ref[...] + b_ref[...]
    a = jnp.maximum(z, 0.0)
    h = jnp.dot(a, w_ref[...], preferred_element_type=jnp.float32)
    hs_o[...] = h * f_ref[...]
    h_o[...] = h


def _tc_last(acc_ref, h_ref, g_ref, s_ref, b_ref, o_ref):
    accp = acc_ref[...]
    o_ref[...] = (g_ref[...] * (accp[0] + accp[1])
                  + s_ref[...] * h_ref[...] + b_ref[...])


_col = jax.ShapeDtypeStruct((NP, 1), jnp.float32)
_mat = jax.ShapeDtypeStruct((NP, D), jnp.float32)

_tc_first_call = pl.pallas_call(
    _tc_first, out_shape=[_mat, _mat, _col, _col, _col])

_tc_mid_call = pl.pallas_call(_tc_mid, out_shape=[_mat, _mat])

_tc_last_call = pl.pallas_call(_tc_last, out_shape=_mat)


@jax.jit
def kernel(x, edge_index, edge_weight, W1, b1, W2, b2, W3, b3):
    src = edge_index[0].astype(jnp.int32)
    dst = edge_index[1].astype(jnp.int32)
    pad_idx = jnp.full((EP - E,), N, jnp.int32)
    src_f = jnp.concatenate([src, pad_idx])
    dst_f = jnp.concatenate([dst, pad_idx])
    src2 = src_f.reshape(NW, EW_)
    dst2 = dst_f.reshape(NW, EW_)

    def split_cores(ids):
        ea = 16 * KA * BLK
        eb = 16 * KB * BLK
        ids_p = jnp.concatenate([ids, jnp.full((ea + eb - E,), N, jnp.int32)])
        part_a = ids_p[:ea].reshape(16, KA, BLK)
        part_b = ids_p[ea:].reshape(16, KB, BLK)
        part_a = jnp.pad(part_a, ((0, 0), (0, KMAX - KA), (0, 0)),
                         constant_values=N)
        part_b = jnp.pad(part_b, ((0, 0), (0, KMAX - KB), (0, 0)),
                         constant_values=N)
        return jnp.concatenate([part_a, part_b], axis=0)

    src3 = split_cores(src)
    dst3 = split_cores(dst)

    ew_p = jnp.pad(edge_weight.astype(jnp.float32), (0, NP - N))
    zrow = jnp.zeros((NP,), jnp.float32)
    z128 = jnp.zeros((RP, D), jnp.float32)

    x_p = jnp.pad(x, ((0, NP - N), (0, 0)))
    W1p = jnp.pad(W1, ((0, 0), (0, D - 50)))
    W2p = jnp.pad(W2, ((0, D - 50), (0, D - 50)))
    W3p = jnp.pad(W3, ((0, D - 50), (0, D - 40)))
    b1p = jnp.pad(b1, (0, D - 50))[None, :]
    b2p = jnp.pad(b2, (0, D - 50))[None, :]
    b3p = jnp.pad(b3, (0, D - 40))[None, :]

    deg_parts = _deg_scatter(ew_p, src2, dst2, zrow)
    hs1, h1, f, g, s = _tc_first_call(deg_parts, ew_p[:, None], x_p, W1p)
    acc1 = _edge_scatter(hs1, src3, dst3, z128)
    hs2, h2 = _tc_mid_call(acc1, h1, f, g, s, b1p, W2p)
    acc2 = _edge_scatter(hs2, src3, dst3, z128)
    hs3, h3 = _tc_mid_call(acc2, h2, f, g, s, b2p, W3p)
    acc3 = _edge_scatter(hs3, src3, dst3, z128)
    outp = _tc_last_call(acc3, h3, g, s, b3p)
    return outp[:N, :40]


# speedup vs baseline: 1.1812x; 1.1812x over previous
import functools

import jax
import jax.numpy as jnp
from jax import lax
from jax.experimental import pallas as pl
from jax.experimental.pallas import tpu as pltpu
from jax.experimental.pallas import tpu_sc as plsc

N = 10000
E = 320000
NP = 10112
NW = 32
BLK = 128
KCH = -(-E // (NW * BLK))
EP = NW * BLK * KCH
EW_ = KCH * BLK

NBLK = -(-E // BLK)
KA = 101
KB = -(-(NBLK - 16 * KA) // 16)
KMAX = max(KA, KB)
RP = NP // 16
D = 128
L = 16

_mesh = plsc.VectorSubcoreMesh(core_axis_name="c", subcore_axis_name="s")


@functools.partial(
    pl.kernel,
    mesh=_mesh,
    out_type=jax.ShapeDtypeStruct((2, NP, D), jnp.float32),
    scratch_types=[
        pltpu.VMEM((KMAX, BLK), jnp.int32),
        pltpu.VMEM((KMAX, BLK), jnp.int32),
        pltpu.VMEM((BLK, D), jnp.float32),
        pltpu.VMEM_SHARED((NP, D), jnp.float32),
        pltpu.SemaphoreType.DMA,
    ],
)
def _edge_scatter(table_hbm, src_hbm, dst_hbm, zeros_hbm, out_hbm,
                  src_v, dst_v, rows_v, acc_sh, sem):
    c = lax.axis_index("c")
    s = lax.axis_index("s")
    w = c * 16 + s
    pltpu.sync_copy(zeros_hbm, acc_sh.at[pl.ds(s * RP, RP)])
    pltpu.sync_copy(src_hbm.at[w], src_v)
    pltpu.sync_copy(dst_hbm.at[w], dst_v)
    plsc.subcore_barrier()

    def body(j, carry):
        pltpu.async_copy(table_hbm.at[src_v.at[j]], rows_v, sem).wait()
        pltpu.sync_copy(rows_v, acc_sh.at[dst_v.at[j]], add=True)
        return carry

    @pl.when(c == 0)
    def _():
        lax.fori_loop(0, KA, body, 0)

    @pl.when(c == 1)
    def _():
        lax.fori_loop(0, KB, body, 0)
    plsc.subcore_barrier()
    pltpu.sync_copy(acc_sh.at[pl.ds(s * RP, RP)],
                    out_hbm.at[c, pl.ds(s * RP, RP)])


@functools.partial(
    pl.kernel,
    mesh=_mesh,
    out_type=jax.ShapeDtypeStruct((NW, NP), jnp.float32),
    scratch_types=[
        pltpu.VMEM((NP,), jnp.float32),
        pltpu.VMEM((NP,), jnp.float32),
        pltpu.VMEM((EW_,), jnp.int32),
        pltpu.VMEM((EW_,), jnp.int32),
    ],
    compiler_params=pltpu.CompilerParams(needs_layout_passes=False),
)
def _deg_scatter(ew_hbm, srcf_hbm, dstf_hbm, zrow_hbm, out_hbm,
                 ew_v, acc_v, src_v, dst_v):
    c = lax.axis_index("c")
    s = lax.axis_index("s")
    w = c * 16 + s
    pltpu.sync_copy(ew_hbm, ew_v)
    pltpu.sync_copy(zrow_hbm, acc_v)
    pltpu.sync_copy(srcf_hbm.at[w], src_v)
    pltpu.sync_copy(dstf_hbm.at[w], dst_v)

    def body(g, carry):
        idx_s = src_v[pl.ds(g * L, L)]
        idx_d = dst_v[pl.ds(g * L, L)]
        vals = plsc.load_gather(ew_v, [idx_s])
        plsc.addupdate_scatter(acc_v, [idx_d], vals)
        return carry

    lax.fori_loop(0, EW_ // L, body, 0)
    pltpu.sync_copy(acc_v, out_hbm.at[w])


def _tc_first(deg_ref, ew_ref, x_ref, w_ref, hs_ref, h_ref, f_ref, g_ref, s_ref):
    deg = 1.0 + jnp.sum(deg_ref[...], axis=0)[:, None]
    dinv = jnp.where(deg > 0, lax.rsqrt(deg), 0.0)
    f = dinv * ew_ref[...]
    h = jnp.dot(x_ref[...], w_ref[...], preferred_element_type=jnp.float32)
    hs_ref[...] = h * f
    h_ref[...] = h
    f_ref[...] = f
    g_ref[...] = dinv
    s_ref[...] = dinv * dinv


def _tc_mid(acc_ref, h_ref, f_ref, g_ref, s_ref, b_ref, w_ref, hs_o, h_o):
    accp = acc_ref[...]
    z = g_ref[...] * (accp[0] + accp[1]) + s_ref[...] * h_ref[...] + b_ref[...]
    a = jnp.maximum(z, 0.0)
    h = jnp.dot(a, w_ref[...], preferred_element_type=jnp.float32)
    hs_o[...] = h * f_ref[...]
    h_o[...] = h


def _tc_last(acc_ref, h_ref, g_ref, s_ref, b_ref, o_ref):
    accp = acc_ref[...]
    o_ref[...] = (g_ref[...] * (accp[0] + accp[1])
                  + s_ref[...] * h_ref[...] + b_ref[...])


_col = jax.ShapeDtypeStruct((NP, 1), jnp.float32)
_mat = jax.ShapeDtypeStruct((NP, D), jnp.float32)

_tc_first_call = pl.pallas_call(
    _tc_first, out_shape=[_mat, _mat, _col, _col, _col])

_tc_mid_call = pl.pallas_call(_tc_mid, out_shape=[_mat, _mat])

_tc_last_call = pl.pallas_call(_tc_last, out_shape=_mat)


@jax.jit
def kernel(x, edge_index, edge_weight, W1, b1, W2, b2, W3, b3):
    src = edge_index[0].astype(jnp.int32)
    dst = edge_index[1].astype(jnp.int32)
    pad_idx = jnp.full((EP - E,), N, jnp.int32)
    src_f = jnp.concatenate([src, pad_idx])
    dst_f = jnp.concatenate([dst, pad_idx])
    src2 = src_f.reshape(NW, EW_)
    dst2 = dst_f.reshape(NW, EW_)

    def split_cores(ids):
        ea = 16 * KA * BLK
        eb = 16 * KB * BLK
        ids_p = jnp.concatenate([ids, jnp.full((ea + eb - E,), N, jnp.int32)])
        part_a = ids_p[:ea].reshape(16, KA, BLK)
        part_b = ids_p[ea:].reshape(16, KB, BLK)
        part_a = jnp.pad(part_a, ((0, 0), (0, KMAX - KA), (0, 0)),
                         constant_values=N)
        part_b = jnp.pad(part_b, ((0, 0), (0, KMAX - KB), (0, 0)),
                         constant_values=N)
        return jnp.concatenate([part_a, part_b], axis=0)

    src3 = split_cores(src)
    dst3 = split_cores(dst)

    ew_p = jnp.pad(edge_weight.astype(jnp.float32), (0, NP - N))
    zrow = jnp.zeros((NP,), jnp.float32)
    z128 = jnp.zeros((RP, D), jnp.float32)

    x_p = jnp.pad(x, ((0, NP - N), (0, 0)))
    W1p = jnp.pad(W1, ((0, 0), (0, D - 50)))
    W2p = jnp.pad(W2, ((0, D - 50), (0, D - 50)))
    W3p = jnp.pad(W3, ((0, D - 50), (0, D - 40)))
    b1p = jnp.pad(b1, (0, D - 50))[None, :]
    b2p = jnp.pad(b2, (0, D - 50))[None, :]
    b3p = jnp.pad(b3, (0, D - 40))[None, :]

    deg_parts = _deg_scatter(ew_p, src2, dst2, zrow)
    hs1, h1, f, g, s = _tc_first_call(deg_parts, ew_p[:, None], x_p, W1p)
    acc1 = _edge_scatter(hs1, src3, dst3, z128)
    hs2, h2 = _tc_mid_call(acc1, h1, f, g, s, b1p, W2p)
    acc2 = _edge_scatter(hs2, src3, dst3, z128)
    hs3, h3 = _tc_mid_call(acc2, h2, f, g, s, b2p, W3p)
    acc3 = _edge_scatter(hs3, src3, dst3, z128)
    outp = _tc_last_call(acc3, h3, g, s, b3p)
    return outp[:N, :40]
